# baseline (speedup 1.0000x reference)
import jax
import jax.numpy as jnp
from jax.experimental import pallas as pl

B, M, D = 4, 8192, 1024
L = 77
H = 16
NMAX = 2560
RHO_MIN, RHO_MAX = 0.05, 0.5
LAM_T, LAM_M, LAM_S = 1.0, 1.7, 0.05


TM_Q = 2048


def _q_body(x_ref, w_ref, o_ref):
    o_ref[...] = jnp.dot(
        x_ref[...].astype(jnp.bfloat16), w_ref[...].astype(jnp.bfloat16),
        preferred_element_type=jnp.float32,
    ).astype(jnp.bfloat16)


def _q_pallas(X_v, Wq):
    x = X_v.reshape(B * M, D)
    q = pl.pallas_call(
        _q_body,
        grid=(B * M // TM_Q,),
        in_specs=[
            pl.BlockSpec((TM_Q, D), lambda i: (i, 0)),
            pl.BlockSpec((D, D), lambda i: (0, 0)),
        ],
        out_specs=pl.BlockSpec((TM_Q, D), lambda i: (i, 0)),
        out_shape=jax.ShapeDtypeStruct((B * M, D), jnp.bfloat16),
    )(x, Wq)
    return q


def kernel(X_v, Q_t, Wq, Wk, Wv, w_s, a_r, b_r, W_re):
    dh = D // H
    q = (X_v @ Wq).reshape(B, M, H, dh)
    k = (Q_t @ Wk).reshape(B, L, H, dh)
    v = (Q_t @ Wv).reshape(B, L, H, dh)
    logits = jnp.einsum("bmhd,blhd->bhml", q, k) / jnp.sqrt(float(dh))
    attn = jax.nn.softmax(logits, axis=-1)
    ctx = jnp.einsum("bhml,blhd->bmhd", attn, v).reshape(B, M, D)
    s = ctx @ w_s
    pooled = jnp.mean(jax.nn.sigmoid(s), axis=1)
    r = jax.nn.sigmoid(a_r * pooled + b_r)
    rho = RHO_MIN + (RHO_MAX - RHO_MIN) * r
    n_vec = jnp.minimum(jnp.round(rho * M), float(NMAX)).astype(jnp.int32)
    k_keep = min(NMAX, M)
    _, top_idx = jax.lax.top_k(s, k_keep)
    Z = jnp.take_along_axis(X_v, top_idx[..., None], axis=1)
    Z = Z @ W_re
    M_f = float(M)
    flops_proxy = (rho * M_f) ** 2 / float(NMAX ** 2)
    kv_proxy = rho * M_f / float(NMAX)
    rho_loss = (rho - jnp.mean(rho)) ** 2
    return (
        Z,
        top_idx,
        rho,
        r,
        n_vec,
        jnp.mean(flops_proxy) * LAM_T,
        jnp.mean(kv_proxy) * LAM_M,
        jnp.mean(rho_loss) * LAM_S,
    )


# speedup vs baseline: 1.3989x; 1.3989x over previous
import functools

import jax
import jax.numpy as jnp
from jax.experimental import pallas as pl
from jax.experimental.pallas import tpu as pltpu
from jax.experimental.pallas import tpu_sc as plsc

B, M, D = 4, 8192, 1024
L = 77
H = 16
NMAX = 2560
RHO_MIN, RHO_MAX = 0.05, 0.5
LAM_T, LAM_M, LAM_S = 1.0, 1.7, 0.05

NTOT = B * NMAX


def _make_sc_gather():
    info = plsc.get_sparse_core_info()
    nw = info.num_cores * info.num_subcores
    rows_per_w = NTOT // nw
    chunk = 64
    nchunk = rows_per_w // chunk
    mesh = plsc.VectorSubcoreMesh(core_axis_name="c", subcore_axis_name="s")

    @functools.partial(
        pl.kernel,
        mesh=mesh,
        out_type=jax.ShapeDtypeStruct((NTOT, D), jnp.float32),
        scratch_types=[
            pltpu.VMEM((nchunk, chunk), jnp.int32),
            pltpu.VMEM((chunk, D), jnp.float32),
            pltpu.SemaphoreType.DMA,
        ],
    )
    def gather_k(x_hbm, idx_hbm, out_hbm, idx_v, rows_v, sem):
        wid = jax.lax.axis_index("s") * info.num_cores + jax.lax.axis_index("c")
        base = wid * rows_per_w
        for ci in range(nchunk):
            pltpu.sync_copy(
                idx_hbm.at[pl.ds(base + ci * chunk, chunk)], idx_v.at[ci])
            pltpu.async_copy(x_hbm.at[idx_v.at[ci]], rows_v, sem).wait()
            pltpu.sync_copy(rows_v, out_hbm.at[pl.ds(base + ci * chunk, chunk)])

    return gather_k


_sc_gather = _make_sc_gather()


_TMZ = 1024


def _z_body(x_ref, w_ref, o_ref):
    o_ref[...] = jnp.dot(
        x_ref[...].astype(jnp.bfloat16),
        w_ref[...].astype(jnp.bfloat16),
        preferred_element_type=jnp.float32,
    )


def _z_pallas(rows, W_re):
    return pl.pallas_call(
        _z_body,
        grid=(NTOT // _TMZ,),
        in_specs=[
            pl.BlockSpec((_TMZ, D), lambda i: (i, 0)),
            pl.BlockSpec((D, D), lambda i: (0, 0)),
        ],
        out_specs=pl.BlockSpec((_TMZ, D), lambda i: (i, 0)),
        out_shape=jax.ShapeDtypeStruct((NTOT, D), jnp.float32),
    )(rows, W_re)


def kernel(X_v, Q_t, Wq, Wk, Wv, w_s, a_r, b_r, W_re):
    dh = D // H
    q = (X_v @ Wq).reshape(B, M, H, dh)
    k = (Q_t @ Wk).reshape(B, L, H, dh)
    v = (Q_t @ Wv).reshape(B, L, H, dh)
    logits = jnp.einsum("bmhd,blhd->bhml", q, k) / jnp.sqrt(float(dh))
    attn = jax.nn.softmax(logits, axis=-1)
    ctx = jnp.einsum("bhml,blhd->bmhd", attn, v).reshape(B, M, D)
    s = ctx @ w_s
    pooled = jnp.mean(jax.nn.sigmoid(s), axis=1)
    r = jax.nn.sigmoid(a_r * pooled + b_r)
    rho = RHO_MIN + (RHO_MAX - RHO_MIN) * r
    n_vec = jnp.minimum(jnp.round(rho * M), float(NMAX)).astype(jnp.int32)
    _, top_idx = jax.lax.top_k(s, NMAX)
    flat_idx = (top_idx + (jnp.arange(B, dtype=jnp.int32) * M)[:, None]).reshape(NTOT)
    rows = _sc_gather(X_v.reshape(B * M, D), flat_idx)
    Z = _z_pallas(rows, W_re).reshape(B, NMAX, D)
    M_f = float(M)
    flops_proxy = (rho * M_f) ** 2 / float(NMAX ** 2)
    kv_proxy = rho * M_f / float(NMAX)
    rho_loss = (rho - jnp.mean(rho)) ** 2
    return (
        Z,
        top_idx,
        rho,
        r,
        n_vec,
        jnp.mean(flops_proxy) * LAM_T,
        jnp.mean(kv_proxy) * LAM_M,
        jnp.mean(rho_loss) * LAM_S,
    )
